# Initial kernel scaffold; baseline (speedup 1.0000x reference)
import jax
import jax.numpy as jnp
from jax.experimental import pallas as pl


def kernel(src, dst, t, x, edge_attr, memory, Wx, bx, Wt, bt, W_ih, b_ih, W_hh, b_hh, Wq, bq, Wk, bk, Wv, bv, We, be, Wskip, bskip, W1, b1, W2, b2):
    raise NotImplementedError("write your pallas kernel here")


# speedup vs baseline: 5.2611x; 5.2611x over previous
import functools

import jax
import jax.numpy as jnp
from jax import lax
from jax.experimental import pallas as pl
from jax.experimental.pallas import tpu as pltpu
from jax.experimental.pallas import tpu_sc as plsc

N = 10000
E = 160000
D_IN = 128
D_EDGE = 16
H = 64

NC = 2
NS = 16
NW = NC * NS
NPAD = 12288
NPW = NPAD // NW
EPW = E // NW

@functools.lru_cache(maxsize=None)
def _mesh():
    return plsc.VectorSubcoreMesh(core_axis_name="c", subcore_axis_name="s")


def _wid():
    return lax.axis_index("s") * NC + lax.axis_index("c")


def _take16(x, idx):
    dn = lax.GatherDimensionNumbers(
        offset_dims=(), collapsed_slice_dims=(0,), start_index_map=(0,))
    return lax.gather(x, idx[:, None], dn, (1,),
                      mode=lax.GatherScatterMode.PROMISE_IN_BOUNDS)


def _chunks(total, step):
    out, off = [], 0
    while off < total:
        sz = min(step, total - off)
        out.append((off, sz))
        off += sz
    return out


def _gather_chunked(table_h, idx_ref, out_ref, sem, n, step):
    descs = [
        pltpu.async_copy(table_h.at[idx_ref.at[pl.ds(off, sz)]],
                         out_ref.at[pl.ds(off, sz)], sem)
        for off, sz in _chunks(n, step)
    ]
    for d in descs:
        d.wait()


def _sc_a1_body(src_h, dst_h, neg1_h, ld_h, ls_h,
                ev, ibd, ibs, lst_sh, ldt_sh, sem):
    w = _wid()
    sid = lax.axis_index("s")
    base = w * EPW
    tbase = sid * NPAD
    iot = lax.iota(jnp.int32, 16)

    pltpu.sync_copy(neg1_h, ldt_sh.at[pl.ds(tbase, NPAD)])
    pltpu.sync_copy(neg1_h, lst_sh.at[pl.ds(tbase, NPAD)])

    def do_chunk(o, sz):
        def bld(v, c):
            ev[pl.ds(v * 16, 16)] = o + ((v * 16 + iot) >> 3)
            return c
        lax.fori_loop(0, (sz * 8) // 16, bld, 0)
        _gather_chunked(dst_h, ev, ibd, sem, sz * 8, _B1C)
        _gather_chunked(src_h, ev, ibs, sem, sz * 8, _B1C)

        def addb(v, c):
            ibd[pl.ds(v * 16, 16)] = ibd[pl.ds(v * 16, 16)] + tbase
            ibs[pl.ds(v * 16, 16)] = ibs[pl.ds(v * 16, 16)] + tbase
            return c
        lax.fori_loop(0, (sz * 8) // 16, addb, 0)

        def ed(e, c):
            e8 = e * 8
            pltpu.sync_copy(ev.at[pl.ds(e8, 1)],
                            ldt_sh.at[ibd.at[pl.ds(e8, 1)]])
            pltpu.sync_copy(ev.at[pl.ds(e8, 1)],
                            lst_sh.at[ibs.at[pl.ds(e8, 1)]])
            return c
        lax.fori_loop(0, sz, ed, 0)

    nfull = EPW // _B1C
    lax.fori_loop(0, nfull, lambda k, c: (do_chunk(base + k * _B1C, _B1C), c)[1],
                  0)
    if EPW % _B1C:
        do_chunk(base + nfull * _B1C, EPW % _B1C)

    pltpu.sync_copy(ldt_sh.at[pl.ds(tbase, NPAD)], ld_h.at[pl.ds(w * NPAD, NPAD)])
    pltpu.sync_copy(lst_sh.at[pl.ds(tbase, NPAD)], ls_h.at[pl.ds(w * NPAD, NPAD)])


def _sc_a2_body(ld_h, ls_h, t_h, src_h, dst_h, mem_h,
           tg_h, role_h, has_h, other_h, m0o_h,
           tbl, ldcb, jjb, roleb, hasb, srcgb, dstgb, otherb, tgb, m0ob, sem):
    w = _wid()
    base = w * NPW

    for f in range(NW):
        pltpu.sync_copy(ld_h.at[pl.ds(f * NPAD + base, NPW)], tbl.at[f])

    def red_body(v, c):
        off = v * 16
        acc = tbl[0, pl.ds(off, 16)]
        for f in range(1, NW):
            acc = jnp.maximum(acc, tbl[f, pl.ds(off, 16)])
        ldcb[pl.ds(off, 16)] = acc
        return c
    lax.fori_loop(0, NPW // 16, red_body, 0)

    for f in range(NW):
        pltpu.sync_copy(ls_h.at[pl.ds(f * NPAD + base, NPW)], tbl.at[f])

    iot16 = lax.iota(jnp.int32, 16)
    one16 = jnp.full((16,), 1, jnp.int32)
    zero16 = jnp.full((16,), 0, jnp.int32)

    def sel_body(v, c):
        off = v * 16
        acc = tbl[0, pl.ds(off, 16)]
        for f in range(1, NW):
            acc = jnp.maximum(acc, tbl[f, pl.ds(off, 16)])
        ldc = ldcb[pl.ds(off, 16)]
        role = ldc >= 0
        jsel = jnp.where(role, ldc, acc)
        nodeid = base + off + iot16
        jsel = jnp.where(nodeid < N, jsel, -1)
        jjb[pl.ds(off, 16)] = jnp.maximum(jsel, 0)
        roleb[pl.ds(off, 16)] = jnp.where(role, one16, zero16)
        hasb[pl.ds(off, 16)] = jnp.where(jsel >= 0, one16, zero16)
        return c
    lax.fori_loop(0, NPW // 16, sel_body, 0)

    _gather_chunked(t_h, jjb, tgb, sem, NPW, _B1C)
    _gather_chunked(src_h, jjb, srcgb, sem, NPW, _B1C)
    _gather_chunked(dst_h, jjb, dstgb, sem, NPW, _B1C)

    def oth_body(v, c):
        off = v * 16
        role = roleb[pl.ds(off, 16)] > 0
        otherb[pl.ds(off, 16)] = jnp.where(
            role, srcgb[pl.ds(off, 16)], dstgb[pl.ds(off, 16)])
        return c
    lax.fori_loop(0, NPW // 16, oth_body, 0)

    _gather_chunked(mem_h, otherb, m0ob, sem, NPW, _B1C)

    pltpu.sync_copy(tgb, tg_h.at[pl.ds(base, NPW)])
    pltpu.sync_copy(roleb, role_h.at[pl.ds(base, NPW)])
    pltpu.sync_copy(hasb, has_h.at[pl.ds(base, NPW)])
    pltpu.sync_copy(otherb, other_h.at[pl.ds(base, NPW)])
    pltpu.sync_copy(m0ob, m0o_h.at[pl.ds(base, NPW)])


def _sc_a3_body(other_h, mem1_h, m1o_h, otherb, rows, sem):
    w = _wid()
    base = w * NPW
    pltpu.sync_copy(other_h.at[pl.ds(base, NPW)], otherb)
    _gather_chunked(mem1_h, otherb, rows, sem, NPW, _B1C)
    pltpu.sync_copy(rows, m1o_h.at[pl.ds(base, NPW)])


_B1C = 128

def _sc_b1_body(src_h, dst_h, q_h, kv_h, qg_h, kvg_h, dstc, srcc, qrows,
                kvrows, sem):
    w = _wid()
    base = w * EPW
    for off, sz in _chunks(EPW, _B1C):
        o = base + off
        di = dstc.at[pl.ds(0, sz)] if sz < _B1C else dstc
        si = srcc.at[pl.ds(0, sz)] if sz < _B1C else srcc
        qr = qrows.at[pl.ds(0, sz)] if sz < _B1C else qrows
        kr = kvrows.at[pl.ds(0, sz)] if sz < _B1C else kvrows
        pltpu.sync_copy(dst_h.at[pl.ds(o, sz)], di)
        pltpu.sync_copy(src_h.at[pl.ds(o, sz)], si)
        d1 = pltpu.async_copy(q_h.at[di], qr, sem)
        d2 = pltpu.async_copy(kv_h.at[si], kr, sem)
        d1.wait()
        d2.wait()
        pltpu.sync_copy(qr, qg_h.at[pl.ds(o, sz)])
        pltpu.sync_copy(kr, kvg_h.at[pl.ds(o, sz)])


def _sc_b2_body(dst_h, ex_h, wt_h, zf_h, z1_h, aggp_h, den_h,
           dstc, exc, wbuf, ib, agg_sh, den_sh):
    cid = lax.axis_index("c")
    sid = lax.axis_index("s")
    w = _wid()
    nrow = NPAD // NS
    nflat = (H * NPAD) // NS

    pltpu.sync_copy(zf_h.at[pl.ds(sid * nflat, nflat)],
                    agg_sh.at[pl.ds(sid * nflat, nflat)])
    pltpu.sync_copy(z1_h.at[pl.ds(sid * nrow, nrow)],
                    den_sh.at[pl.ds(sid * nrow, nrow)])
    plsc.subcore_barrier()

    nchunk = E // _B1C
    nk = jnp.where(w < nchunk - (nchunk // NW) * NW, nchunk // NW + 1,
                   nchunk // NW)

    def chunk_body(k, c0):
        o = (k * NW + w) * _B1C
        pltpu.sync_copy(dst_h.at[pl.ds(o, _B1C)], dstc)
        pltpu.sync_copy(ex_h.at[pl.ds(o, _B1C)], exc)
        pltpu.sync_copy(exc, den_sh.at[dstc], add=True)
        pltpu.sync_copy(wt_h.at[:, pl.ds(o, _B1C)], wbuf)

        def hrow(h, c):
            hoff = h * NPAD

            def grp(g, c2):
                gi = g * 16
                ib[pl.ds(gi, 16)] = dstc[pl.ds(gi, 16)] + hoff
                return c2
            lax.fori_loop(0, _B1C // 16, grp, 0)
            pltpu.sync_copy(wbuf.at[h], agg_sh.at[ib], add=True)
            return c
        lax.fori_loop(0, H, hrow, 0)
        return c0
    lax.fori_loop(0, nk, chunk_body, 0)

    plsc.subcore_barrier()
    pltpu.sync_copy(agg_sh.at[pl.ds(sid * nflat, nflat)],
                    aggp_h.at[cid, pl.ds(sid * nflat, nflat)])
    pltpu.sync_copy(den_sh.at[pl.ds(sid * nrow, nrow)],
                    den_h.at[cid, pl.ds(sid * nrow, nrow)])


def _sc_b3_body(dst_h, fv_h, out_h, dstc, rows, sem):
    w = _wid()
    base = w * EPW
    for off, sz in _chunks(EPW, _B1C):
        o = base + off
        di = dstc.at[pl.ds(0, sz)] if sz < _B1C else dstc
        rr = rows.at[pl.ds(0, sz)] if sz < _B1C else rows
        pltpu.sync_copy(dst_h.at[pl.ds(o, sz)], di)
        pltpu.async_copy(fv_h.at[di], rr, sem).wait()
        pltpu.sync_copy(rr, out_h.at[pl.ds(o, sz)])


@functools.lru_cache(maxsize=None)
def _sc_kernels():
    mesh = _mesh()
    f32, i32 = jnp.float32, jnp.int32
    sds = jax.ShapeDtypeStruct

    def mk(body, out_type, scratch):
        return pl.kernel(body, out_type=out_type, mesh=mesh,
                         scratch_types=scratch)

    return {
        "a1": mk(_sc_a1_body,
                 (sds((NW * NPAD,), i32), sds((NW * NPAD,), i32)),
                 [pltpu.VMEM((8 * _B1C,), i32), pltpu.VMEM((8 * _B1C,), i32),
                  pltpu.VMEM((8 * _B1C,), i32),
                  pltpu.VMEM_SHARED((NS * NPAD,), i32),
                  pltpu.VMEM_SHARED((NS * NPAD,), i32),
                  pltpu.SemaphoreType.DMA]),
        "a2": mk(_sc_a2_body,
                 (sds((NPAD,), f32), sds((NPAD,), i32), sds((NPAD,), i32),
                  sds((NPAD,), i32), sds((NPAD, 2 * H), f32)),
                 [pltpu.VMEM((NW, NPW), i32)] +
                 [pltpu.VMEM((NPW,), i32)] * 7 +
                 [pltpu.VMEM((NPW,), f32), pltpu.VMEM((NPW, 2 * H), f32),
                  pltpu.SemaphoreType.DMA]),
        "a3": mk(_sc_a3_body, sds((NPAD, 2 * H), f32),
                 [pltpu.VMEM((NPW,), i32), pltpu.VMEM((NPW, 2 * H), f32),
                  pltpu.SemaphoreType.DMA]),
        "b1": mk(_sc_b1_body,
                 (sds((E, 2 * H), f32), sds((E, 2 * H), f32)),
                 [pltpu.VMEM((_B1C,), i32), pltpu.VMEM((_B1C,), i32),
                  pltpu.VMEM((_B1C, 2 * H), f32), pltpu.VMEM((_B1C, 2 * H), f32),
                  pltpu.SemaphoreType.DMA]),
        "b2": mk(_sc_b2_body,
                 (sds((NC, H * NPAD), f32), sds((NC, NPAD), f32)),
                 [pltpu.VMEM((_B1C,), i32), pltpu.VMEM((_B1C,), f32),
                  pltpu.VMEM((H, _B1C), f32), pltpu.VMEM((_B1C,), i32),
                  pltpu.VMEM_SHARED((H * NPAD,), f32),
                  pltpu.VMEM_SHARED((NPAD,), f32)]),
        "b3": mk(_sc_b3_body, sds((E,), f32),
                 [pltpu.VMEM((_B1C,), i32), pltpu.VMEM((_B1C,), f32),
                  pltpu.SemaphoreType.DMA]),
    }


def _sc_a1(*a):
    return _sc_kernels()["a1"](*a)


def _sc_a2(*a):
    return _sc_kernels()["a2"](*a)


def _sc_a3(*a):
    return _sc_kernels()["a3"](*a)


def _sc_b1(*a):
    return _sc_kernels()["b1"](*a)


def _sc_b2(*a):
    return _sc_kernels()["b2"](*a)


def _sc_b3(*a):
    return _sc_kernels()["b3"](*a)


_NBLK = 1536
_EBLK = 4000


def _full(shape):
    return pl.BlockSpec(shape, lambda i: tuple(0 for _ in shape))


def _tc_pre_n_body(x, Wx, bx, Wq, bq, Wk, bk, Wv, bv, Wsk, bsk, q_o, kv_o, hsk_o):
    hx = jnp.dot(x[...], Wx[...]) + bx[...]
    q = jnp.dot(hx, Wq[...]) + bq[...]
    q_o[...] = jnp.concatenate([q, jnp.zeros_like(q)], axis=1)
    k = jnp.dot(hx, Wk[...]) + bk[...]
    v = jnp.dot(hx, Wv[...]) + bv[...]
    kv_o[...] = jnp.concatenate([k, v], axis=1)
    hsk_o[...] = jnp.dot(hx, Wsk[...]) + bsk[...]


def _tc_pre_n(x_p, Wx, bx, Wq, bq, Wk, bk, Wv, bv, Wsk, bsk):
    g = NPAD // _NBLK
    return pl.pallas_call(
        _tc_pre_n_body,
        grid=(g,),
        in_specs=[pl.BlockSpec((_NBLK, D_IN), lambda i: (i, 0)),
                  _full((D_IN, H)), _full((1, H)),
                  _full((H, H)), _full((1, H)),
                  _full((H, H)), _full((1, H)),
                  _full((H, H)), _full((1, H)),
                  _full((H, H)), _full((1, H))],
        out_specs=[pl.BlockSpec((_NBLK, 2 * H), lambda i: (i, 0)),
                   pl.BlockSpec((_NBLK, 2 * H), lambda i: (i, 0)),
                   pl.BlockSpec((_NBLK, H), lambda i: (i, 0))],
        out_shape=[jax.ShapeDtypeStruct((NPAD, 2 * H), jnp.float32),
                   jax.ShapeDtypeStruct((NPAD, 2 * H), jnp.float32),
                   jax.ShapeDtypeStruct((NPAD, H), jnp.float32)],
    )(x_p, Wx, bx, Wq, bq, Wk, bk, Wv, bv, Wsk, bsk)


def _tc_pre_e_body(ea, We, be, e_o):
    e_o[...] = jnp.dot(ea[...], We[...]) + be[...]


def _tc_pre_e(ea, We, be):
    return pl.pallas_call(
        _tc_pre_e_body,
        grid=(E // _EBLK,),
        in_specs=[pl.BlockSpec((_EBLK, D_EDGE), lambda i: (i, 0)),
                  _full((D_EDGE, H)), _full((1, H))],
        out_specs=pl.BlockSpec((_EBLK, H), lambda i: (i, 0)),
        out_shape=jax.ShapeDtypeStruct((E, H), jnp.float32),
    )(ea, We, be)


def _tc_b1_body(qg, kvg, e, ex_o, wpre_o):
    qgv = qg[:, :H]
    k = kvg[:, :H] + e[...]
    v = kvg[:, H:] + e[...]
    alpha = jnp.sum(qgv * k, axis=1, keepdims=True) * (1.0 / 8.0)
    ex = jnp.exp(alpha)
    ex_o[...] = ex
    wpre_o[...] = ex * v


def _tc_b1(qg, kvg, e):
    return pl.pallas_call(
        _tc_b1_body,
        grid=(E // _EBLK,),
        in_specs=[pl.BlockSpec((_EBLK, 2 * H), lambda i: (i, 0)),
                  pl.BlockSpec((_EBLK, 2 * H), lambda i: (i, 0)),
                  pl.BlockSpec((_EBLK, H), lambda i: (i, 0))],
        out_specs=[pl.BlockSpec((_EBLK, 1), lambda i: (i, 0)),
                   pl.BlockSpec((_EBLK, H), lambda i: (i, 0))],
        out_shape=[jax.ShapeDtypeStruct((E, 1), jnp.float32),
                   jax.ShapeDtypeStruct((E, H), jnp.float32)],
    )(qg, kvg, e)


def _tc_b2_body(aggp, den, hsk, W1, b1, W2, b2, fv_o):
    s = aggp[0] + aggp[1]
    d = den[0] + den[1] + 1e-16
    agg = s / d[:, None]
    h_local = agg + hsk[...]
    hl8 = jnp.dot(h_local, W1[...]) + b1[...]
    fv_o[...] = jnp.dot(jnp.maximum(hl8, 0.0), W2[...]) + b2[...]


def _tc_b2(aggp, den, hsk, W1, b1, W2, b2):
    g = NPAD // _NBLK
    return pl.pallas_call(
        _tc_b2_body,
        grid=(g,),
        in_specs=[pl.BlockSpec((NC, _NBLK, H), lambda i: (0, i, 0)),
                  pl.BlockSpec((NC, _NBLK), lambda i: (0, i)),
                  pl.BlockSpec((_NBLK, H), lambda i: (i, 0)),
                  _full((H, 8)), _full((1, 8)),
                  _full((8, 1)), _full((1, 1))],
        out_specs=pl.BlockSpec((_NBLK, 1), lambda i: (i, 0)),
        out_shape=jax.ShapeDtypeStruct((NPAD, 1), jnp.float32),
    )(aggp, den, hsk, W1, b1, W2, b2)


def _gru_math(msg, h, W_ih, b_ih, W_hh, b_hh):
    gi = jnp.dot(msg, W_ih) + b_ih
    gh = jnp.dot(h, W_hh) + b_hh
    r = jax.nn.sigmoid(gi[:, :H] + gh[:, :H])
    z = jax.nn.sigmoid(gi[:, H:2 * H] + gh[:, H:2 * H])
    n = jnp.tanh(gi[:, 2 * H:] + r * gh[:, 2 * H:])
    return (1.0 - z) * n + z * h


def _tc_gru1_body(mem, m0o, tg, role, has, Wt, bt, W_ih, b_ih, W_hh, b_hh,
                  mem1_o, raw_o):
    m0 = mem[:, :H]
    m0o_v = m0o[:, :H]
    c = jnp.cos(jnp.dot(tg[...], Wt[...]) + bt[...])
    role_v = role[...] > 0
    raw = jnp.where(role_v, m0o_v, m0) + c
    raw_o[...] = raw
    msg = jnp.concatenate([m0, m0o_v, raw, c], axis=1)
    g = _gru_math(msg, m0, W_ih[...], b_ih[...], W_hh[...], b_hh[...])
    m1 = jnp.where(has[...] > 0, g, m0)
    mem1_o[...] = jnp.concatenate([m1, jnp.zeros_like(m1)], axis=1)


def _tc_gru1(mem_p, m0o, tg, role, has, Wt, bt, W_ih, b_ih, W_hh, b_hh):
    g = NPAD // _NBLK
    return pl.pallas_call(
        _tc_gru1_body,
        grid=(g,),
        in_specs=[pl.BlockSpec((_NBLK, 2 * H), lambda i: (i, 0)),
                  pl.BlockSpec((_NBLK, 2 * H), lambda i: (i, 0)),
                  pl.BlockSpec((_NBLK, 1), lambda i: (i, 0)),
                  pl.BlockSpec((_NBLK, 1), lambda i: (i, 0)),
                  pl.BlockSpec((_NBLK, 1), lambda i: (i, 0)),
                  _full((1, H)), _full((1, H)),
                  _full((4 * H, 3 * H)), _full((1, 3 * H)),
                  _full((H, 3 * H)), _full((1, 3 * H))],
        out_specs=[pl.BlockSpec((_NBLK, 2 * H), lambda i: (i, 0)),
                   pl.BlockSpec((_NBLK, H), lambda i: (i, 0))],
        out_shape=[jax.ShapeDtypeStruct((NPAD, 2 * H), jnp.float32),
                   jax.ShapeDtypeStruct((NPAD, H), jnp.float32)],
    )(mem_p, m0o, tg, role, has, Wt, bt, W_ih, b_ih, W_hh, b_hh)


def _tc_gru2_body(mem1, m1o, raw, has, bt, W_ih, b_ih, W_hh, b_hh, mem2_o):
    m1 = mem1[:, :H]
    enc2 = jnp.cos(bt[...])
    msg = jnp.concatenate(
        [m1, m1o[:, :H], raw[...], jnp.broadcast_to(enc2, m1.shape)], axis=1)
    g = _gru_math(msg, m1, W_ih[...], b_ih[...], W_hh[...], b_hh[...])
    mem2_o[...] = jnp.where(has[...] > 0, g, m1)


def _tc_gru2(mem1, m1o, raw, has, bt, W_ih, b_ih, W_hh, b_hh):
    g = NPAD // _NBLK
    return pl.pallas_call(
        _tc_gru2_body,
        grid=(g,),
        in_specs=[pl.BlockSpec((_NBLK, 2 * H), lambda i: (i, 0)),
                  pl.BlockSpec((_NBLK, 2 * H), lambda i: (i, 0)),
                  pl.BlockSpec((_NBLK, H), lambda i: (i, 0)),
                  pl.BlockSpec((_NBLK, 1), lambda i: (i, 0)),
                  _full((1, H)),
                  _full((4 * H, 3 * H)), _full((1, 3 * H)),
                  _full((H, 3 * H)), _full((1, 3 * H))],
        out_specs=pl.BlockSpec((_NBLK, H), lambda i: (i, 0)),
        out_shape=jax.ShapeDtypeStruct((NPAD, H), jnp.float32),
    )(mem1, m1o, raw, has, bt, W_ih, b_ih, W_hh, b_hh)


def kernel(src, dst, t, x, edge_attr, memory, Wx, bx, Wt, bt, W_ih, b_ih,
           W_hh, b_hh, Wq, bq, Wk, bk, Wv, bv, We, be, Wskip, bskip,
           W1, b1, W2, b2):
    pad_n = NPAD - N
    x_p = jnp.pad(x, ((0, pad_n), (0, 0)))
    mem_p = jnp.pad(memory, ((0, pad_n), (0, H)))
    zf = jnp.zeros((H * NPAD,), jnp.float32)
    z1 = jnp.zeros((NPAD,), jnp.float32)
    r2 = lambda b: b.reshape(1, -1)

    q, kv, hskip = _tc_pre_n(x_p, Wx, r2(bx), Wq, r2(bq), Wk, r2(bk),
                             Wv, r2(bv), Wskip, r2(bskip))
    e = _tc_pre_e(edge_attr, We, r2(be))

    neg1 = jnp.full((NPAD,), -1, jnp.int32)
    ldtab, lstab = _sc_a1(src, dst, neg1)
    tg, role, has, other, m0o = _sc_a2(ldtab, lstab, t, src, dst, mem_p)
    mem1, rawsel = _tc_gru1(mem_p, m0o, tg[:, None], role[:, None],
                            has[:, None], Wt, r2(bt), W_ih, r2(b_ih),
                            W_hh, r2(b_hh))
    m1o = _sc_a3(other, mem1)
    mem2 = _tc_gru2(mem1, m1o, rawsel, has[:, None], r2(bt), W_ih, r2(b_ih),
                    W_hh, r2(b_hh))

    qg, kvg = _sc_b1(src, dst, q, kv)
    ex, wpre = _tc_b1(qg, kvg, e)
    aggp, den = _sc_b2(dst, ex.reshape(E), wpre.T, zf, z1)
    aggt = aggp.reshape(NC, H, NPAD).transpose(0, 2, 1)
    fv = _tc_b2(aggt, den, hskip, W1, r2(b1), W2, r2(b2))
    out = _sc_b3(dst, fv.reshape(NPAD))

    return out, mem2[:N]
